# Initial kernel scaffold; baseline (speedup 1.0000x reference)
import jax
import jax.numpy as jnp
from jax.experimental import pallas as pl


def kernel(coord1, coord2, A_list):
    raise NotImplementedError("write your pallas kernel here")


# speedup vs baseline: 192.0733x; 192.0733x over previous
import functools

import jax
import jax.numpy as jnp
from jax import lax
from jax.experimental import pallas as pl
from jax.experimental.pallas import tpu as pltpu
from jax.experimental.pallas import tpu_sc as plsc

NTILES = 32


def _pack_body(c1x, c1y, c1z, c2x, c2y, c2z, dx, dy, dz, xy_ref):
    x = c2x[...] - c1x[...]
    y = c2y[...] - c1y[...]
    z = c2z[...] - c1z[...]
    dx[...] = x
    dy[...] = y
    dz[...] = z

    def rne_hi(v):
        u = lax.bitcast_convert_type(v, jnp.uint32)
        r = u + jnp.uint32(0x7FFF) + ((u >> 16) & jnp.uint32(1))
        return r & jnp.uint32(0xFFFF0000)

    xy_ref[...] = lax.bitcast_convert_type(
        rne_hi(x) | (rne_hi(y) >> 16), jnp.int32
    )


def _make_sc_kernel(B, N, E, BLK):
    NBLK = N // BLK
    CPB = BLK // 16
    TPB = NTILES // B
    mesh = plsc.VectorSubcoreMesh(
        core_axis_name="c", subcore_axis_name="s", num_cores=2, num_subcores=16
    )

    @functools.partial(
        pl.kernel,
        out_type=jax.ShapeDtypeStruct((NTILES, 16), jnp.float32),
        mesh=mesh,
        compiler_params=pltpu.CompilerParams(needs_layout_passes=False),
        scratch_types=[
            pltpu.VMEM((N,), jnp.int32),
            pltpu.VMEM((N,), jnp.float32),
            pltpu.VMEM((BLK * E,), jnp.int32),
            pltpu.VMEM((BLK,), jnp.float32),
            pltpu.VMEM((BLK,), jnp.float32),
            pltpu.VMEM((BLK,), jnp.float32),
            pltpu.VMEM((16,), jnp.float32),
        ],
    )
    def sc_kernel(dx_h, dy_h, dz_h, xy_hbm, a_hbm, out_hbm,
                  txy, tz, abuf, bdx, bdy, bdz, obuf):
        wid = lax.axis_index("s") * 2 + lax.axis_index("c")
        b = wid // TPB
        t = wid % TPB
        pltpu.sync_copy(xy_hbm.at[pl.ds(b * N, N)], txy)
        pltpu.sync_copy(dz_h.at[pl.ds(b * N, N)], tz)
        nblk = (NBLK - t + TPB - 1) // TPB
        iot = lax.iota(jnp.int32, 16)

        def blk_body(k, acc):
            base = (t + k * TPB) * BLK
            g = b * N + base
            pltpu.sync_copy(a_hbm.at[pl.ds(g * E, BLK * E)], abuf)
            for src, dst in ((dx_h, bdx), (dy_h, bdy), (dz_h, bdz)):
                pltpu.sync_copy(src.at[pl.ds(g, BLK)], dst)
            for ch in range(CPB):
                off = ch * 16
                rowb = (off + iot) * E
                ax = jnp.zeros((16,), jnp.float32)
                ay = jnp.zeros((16,), jnp.float32)
                az = jnp.zeros((16,), jnp.float32)
                cnt = jnp.zeros((16,), jnp.float32)
                for e in range(E):
                    idx = plsc.load_gather(abuf, [rowb + e])
                    m = idx >= 0
                    ic = jnp.where(m, idx, 0)
                    w = plsc.load_gather(txy, [ic])
                    x = lax.bitcast_convert_type(
                        w & jnp.int32(-0x10000), jnp.float32
                    )
                    y = lax.bitcast_convert_type(w << 16, jnp.float32)
                    z = plsc.load_gather(tz, [ic])
                    ax = ax + jnp.where(m, x, 0.0)
                    ay = ay + jnp.where(m, y, 0.0)
                    az = az + jnp.where(m, z, 0.0)
                    cnt = cnt + jnp.where(m, 1.0, 0.0)
                inv = 1.0 / cnt
                rx = bdx[pl.ds(off, 16)] - ax * inv
                ry = bdy[pl.ds(off, 16)] - ay * inv
                rz = bdz[pl.ds(off, 16)] - az * inv
                acc = acc + (rx * rx + ry * ry + rz * rz)
            return acc

        total = lax.fori_loop(0, nblk, blk_body, jnp.zeros((16,), jnp.float32))
        obuf[...] = total * (1.0 / (B * 3))
        pltpu.sync_copy(obuf, out_hbm.at[wid])

    return sc_kernel


@functools.lru_cache(maxsize=None)
def _pipeline(B, N, D, E):
    BN = B * N
    BLK = 400
    sc = _make_sc_kernel(B, N, E, BLK)
    vec = jax.ShapeDtypeStruct((BN,), jnp.float32)
    pack = pl.pallas_call(
        _pack_body,
        out_shape=(vec, vec, vec, jax.ShapeDtypeStruct((BN,), jnp.int32)),
    )

    def run(coord1, coord2, A_list):
        c1t = coord1.transpose(2, 0, 1).reshape(3, BN)
        c2t = coord2.transpose(2, 0, 1).reshape(3, BN)
        a_flat = A_list.reshape(BN * E)
        dx, dy, dz, xy_pack = pack(c1t[0], c1t[1], c1t[2],
                                   c2t[0], c2t[1], c2t[2])
        partials = sc(dx, dy, dz, xy_pack, a_flat)
        return jnp.sum(partials)

    return run


def kernel(coord1, coord2, A_list):
    B, N, D = coord1.shape
    E = A_list.shape[-1]
    return _pipeline(B, N, D, E)(coord1, coord2, A_list)
